# Initial kernel scaffold; baseline (speedup 1.0000x reference)
import jax
import jax.numpy as jnp
from jax.experimental import pallas as pl


def kernel(x, edge_index, edge_weight, W_e, b_e, W_d, b_d):
    raise NotImplementedError("write your pallas kernel here")


# speedup vs baseline: 6.0386x; 6.0386x over previous
import functools

import jax
import jax.numpy as jnp
from jax import lax
from jax.experimental import pallas as pl
from jax.experimental.pallas import tpu as pltpu
from jax.experimental.pallas import tpu_sc as plsc

N = 10000
E = 320000
D = 128
H = 32

CHUNK = 128
NCHUNK = E // CHUNK
NW = 32
HW = H // 2


def _encode_body(x_ref, we_ref, be_ref, wd_ref, bd_ref, ybf_ref, l1_ref):
    x = x_ref[...]
    y = jnp.dot(x, we_ref[...], preferred_element_type=jnp.float32)
    y = y + be_ref[...]
    ybf_ref[...] = y.astype(jnp.bfloat16)
    xh = jnp.dot(y, wd_ref[...], preferred_element_type=jnp.float32)
    xh = xh + bd_ref[...]
    r = x - xh
    l1_ref[0, 0] = jnp.sum(r * r)


def _encode(x, W_e, b_e, W_d, b_d):
    return pl.pallas_call(
        _encode_body,
        out_shape=(
            jax.ShapeDtypeStruct((N, H), jnp.bfloat16),
            jax.ShapeDtypeStruct((1, 1), jnp.float32),
        ),
        in_specs=[
            pl.BlockSpec(memory_space=pltpu.VMEM),
            pl.BlockSpec(memory_space=pltpu.VMEM),
            pl.BlockSpec(memory_space=pltpu.VMEM),
            pl.BlockSpec(memory_space=pltpu.VMEM),
            pl.BlockSpec(memory_space=pltpu.VMEM),
        ],
        out_specs=(
            pl.BlockSpec(memory_space=pltpu.VMEM),
            pl.BlockSpec(memory_space=pltpu.SMEM),
        ),
    )(x, W_e, b_e.reshape(1, H), W_d, b_d.reshape(1, D))


def _sc_body(y_hbm, row_hbm, col_hbm, out_hbm,
             idx1_v, idx2_v, rows1_v, rows2_v, sq_v, sem1, sem2):
    cid = lax.axis_index("c")
    sid = lax.axis_index("s")
    wid = sid * 2 + cid
    iota16 = lax.iota(jnp.int32, 16)
    hi_mask = jnp.full((16,), -65536, jnp.int32)

    ntrips = (NCHUNK - 1 - wid) // NW + 1

    def chunk_body(t, carry):
        base = (wid + t * NW) * CHUNK
        pltpu.sync_copy(row_hbm.at[pl.ds(base, CHUNK)], idx1_v)
        pltpu.sync_copy(col_hbm.at[pl.ds(base, CHUNK)], idx2_v)
        cp1 = pltpu.async_copy(y_hbm.at[idx1_v], rows1_v, sem1)
        cp2 = pltpu.async_copy(y_hbm.at[idx2_v], rows2_v, sem2)
        cp1.wait()
        cp2.wait()
        for g in range(CHUNK // 16):
            e_idx = iota16 + g * 16
            acc = jnp.zeros((16,), jnp.float32)
            for d2 in range(HW):
                d_vec = jnp.full((16,), d2, jnp.int32)
                v1 = plsc.load_gather(rows1_v, [e_idx, d_vec])
                v2 = plsc.load_gather(rows2_v, [e_idx, d_vec])
                a1 = plsc.bitcast(lax.shift_left(v1, 16), jnp.float32)
                a2 = plsc.bitcast(lax.shift_left(v2, 16), jnp.float32)
                b1 = plsc.bitcast(lax.bitwise_and(v1, hi_mask), jnp.float32)
                b2 = plsc.bitcast(lax.bitwise_and(v2, hi_mask), jnp.float32)
                da = a1 - a2
                db = b1 - b2
                acc = acc + da * da + db * db
            sq_v[pl.ds(g * 16, 16)] = acc
        pltpu.sync_copy(sq_v, out_hbm.at[pl.ds(base, CHUNK)])
        return carry

    lax.fori_loop(0, ntrips, chunk_body, 0)


_sc_edge_sq = functools.partial(
    pl.kernel,
    out_type=jax.ShapeDtypeStruct((E,), jnp.float32),
    mesh=plsc.VectorSubcoreMesh(core_axis_name="c", subcore_axis_name="s"),
    scratch_types=[
        pltpu.VMEM((CHUNK,), jnp.int32),
        pltpu.VMEM((CHUNK,), jnp.int32),
        pltpu.VMEM((CHUNK, HW), jnp.int32),
        pltpu.VMEM((CHUNK, HW), jnp.int32),
        pltpu.VMEM((CHUNK,), jnp.float32),
        pltpu.SemaphoreType.DMA,
        pltpu.SemaphoreType.DMA,
    ],
    compiler_params=pltpu.CompilerParams(
        needs_layout_passes=False, use_tc_tiling_on_sc=False),
)(_sc_body)


def _finalize_body(sq_ref, w_ref, l1_ref, out_ref):
    dist = jnp.sqrt(sq_ref[...] + 1e-12)
    s2 = jnp.sum(dist * w_ref[...])
    out_ref[0] = l1_ref[0, 0] * (1.0 / (N * D)) + s2 * (1.0 / E)


def _finalize(sq, w, l1):
    return pl.pallas_call(
        _finalize_body,
        out_shape=jax.ShapeDtypeStruct((1,), jnp.float32),
        in_specs=[
            pl.BlockSpec(memory_space=pltpu.VMEM),
            pl.BlockSpec(memory_space=pltpu.VMEM),
            pl.BlockSpec(memory_space=pltpu.SMEM),
        ],
        out_specs=pl.BlockSpec(memory_space=pltpu.SMEM),
    )(sq.reshape(NCHUNK, CHUNK), w.reshape(NCHUNK, CHUNK), l1)


def kernel(x, edge_index, edge_weight, W_e, b_e, W_d, b_d):
    ybf, l1 = _encode(x, W_e, b_e, W_d, b_d)
    y_packed = lax.bitcast_convert_type(ybf.reshape(N, HW, 2), jnp.int32)
    row = edge_index[0]
    col = edge_index[1]
    sq = _sc_edge_sq(y_packed, row, col)
    out = _finalize(sq, edge_weight, l1)
    return out[0]


# speedup vs baseline: 12.7104x; 2.1049x over previous
import functools

import jax
import jax.numpy as jnp
from jax import lax
from jax.experimental import pallas as pl
from jax.experimental.pallas import tpu as pltpu
from jax.experimental.pallas import tpu_sc as plsc

N = 10000
E = 320000
D = 128
H = 32

CHUNK = 128
NCHUNK = E // CHUNK
NW = 32
HW = H // 2


def _encode_body(x_ref, we_ref, be_ref, wd_ref, bd_ref, ybf_ref, l1_ref):
    x = x_ref[...]
    y = jnp.dot(x, we_ref[...], preferred_element_type=jnp.float32)
    y = y + be_ref[...]
    ybf_ref[...] = y.astype(jnp.bfloat16)
    xh = jnp.dot(y, wd_ref[...], preferred_element_type=jnp.float32)
    xh = xh + bd_ref[...]
    r = x - xh
    l1_ref[0, 0] = jnp.sum(r * r)


def _encode(x, W_e, b_e, W_d, b_d):
    return pl.pallas_call(
        _encode_body,
        out_shape=(
            jax.ShapeDtypeStruct((N, H), jnp.bfloat16),
            jax.ShapeDtypeStruct((1, 1), jnp.float32),
        ),
        in_specs=[
            pl.BlockSpec(memory_space=pltpu.VMEM),
            pl.BlockSpec(memory_space=pltpu.VMEM),
            pl.BlockSpec(memory_space=pltpu.VMEM),
            pl.BlockSpec(memory_space=pltpu.VMEM),
            pl.BlockSpec(memory_space=pltpu.VMEM),
        ],
        out_specs=(
            pl.BlockSpec(memory_space=pltpu.VMEM),
            pl.BlockSpec(memory_space=pltpu.SMEM),
        ),
    )(x, W_e, b_e.reshape(1, H), W_d, b_d.reshape(1, D))


EPW = E // NW
SUPER = 512
NSUP = -(-EPW // SUPER)


def _sc_body(y_hbm, row_hbm, col_hbm, out_hbm,
             idxr_v, idxc_v, sq_v, ra0, rb0, ra1, rb1, sem0, sem1):
    cid = lax.axis_index("c")
    sid = lax.axis_index("s")
    wid = sid * 2 + cid
    w0 = wid * EPW
    iota16 = lax.iota(jnp.int32, 16)
    hi_mask = jnp.full((16,), -65536, jnp.int32)

    pltpu.sync_copy(row_hbm.at[pl.ds(w0, EPW)], idxr_v)
    pltpu.sync_copy(col_hbm.at[pl.ds(w0, EPW)], idxc_v)

    bufs = ((ra0, rb0, sem0), (ra1, rb1, sem1))

    def loc_of(t):
        return lax.min(t * SUPER, EPW - SUPER)

    def issue(t, b):
        r1, r2, sem = bufs[b]
        loc = loc_of(t)
        for j in range(SUPER // CHUNK):
            o = j * CHUNK
            pltpu.async_copy(
                y_hbm.at[idxr_v.at[pl.ds(loc + o, CHUNK)]],
                r1.at[pl.ds(o, CHUNK)], sem)
            pltpu.async_copy(
                y_hbm.at[idxc_v.at[pl.ds(loc + o, CHUNK)]],
                r2.at[pl.ds(o, CHUNK)], sem)

    def drain(b):
        r1, r2, sem = bufs[b]
        for j in range(SUPER // CHUNK):
            o = j * CHUNK
            pltpu.make_async_copy(
                y_hbm.at[idxr_v.at[pl.ds(o, CHUNK)]],
                r1.at[pl.ds(o, CHUNK)], sem).wait()
            pltpu.make_async_copy(
                y_hbm.at[idxc_v.at[pl.ds(o, CHUNK)]],
                r2.at[pl.ds(o, CHUNK)], sem).wait()

    def compute(t, b):
        r1, r2, _ = bufs[b]
        loc = loc_of(t)

        def group(g, carry):
            e_idx = iota16 + g * 16
            acc = jnp.zeros((16,), jnp.float32)
            for d2 in range(HW):
                d_vec = jnp.full((16,), d2, jnp.int32)
                v1 = plsc.load_gather(r1, [e_idx, d_vec])
                v2 = plsc.load_gather(r2, [e_idx, d_vec])
                a1 = plsc.bitcast(lax.shift_left(v1, 16), jnp.float32)
                a2 = plsc.bitcast(lax.shift_left(v2, 16), jnp.float32)
                b1 = plsc.bitcast(lax.bitwise_and(v1, hi_mask), jnp.float32)
                b2 = plsc.bitcast(lax.bitwise_and(v2, hi_mask), jnp.float32)
                da = a1 - a2
                db = b1 - b2
                acc = acc + da * da + db * db
            sq_v[pl.ds(loc + g * 16, 16)] = acc
            return carry

        lax.fori_loop(0, SUPER // 16, group, 0)

    issue(0, 0)

    def step(p, carry):
        t0 = p * 2
        issue(t0 + 1, 1)
        drain(0)
        compute(t0, 0)

        @pl.when(t0 + 2 < NSUP)
        def _():
            issue(t0 + 2, 0)

        drain(1)
        compute(t0 + 1, 1)
        return carry

    lax.fori_loop(0, NSUP // 2, step, 0)
    pltpu.sync_copy(sq_v, out_hbm.at[pl.ds(w0, EPW)])


_sc_edge_sq = functools.partial(
    pl.kernel,
    out_type=jax.ShapeDtypeStruct((E,), jnp.float32),
    mesh=plsc.VectorSubcoreMesh(core_axis_name="c", subcore_axis_name="s"),
    scratch_types=[
        pltpu.VMEM((EPW,), jnp.int32),
        pltpu.VMEM((EPW,), jnp.int32),
        pltpu.VMEM((EPW,), jnp.float32),
        pltpu.VMEM((SUPER, HW), jnp.int32),
        pltpu.VMEM((SUPER, HW), jnp.int32),
        pltpu.VMEM((SUPER, HW), jnp.int32),
        pltpu.VMEM((SUPER, HW), jnp.int32),
        pltpu.SemaphoreType.DMA,
        pltpu.SemaphoreType.DMA,
    ],
    compiler_params=pltpu.CompilerParams(
        needs_layout_passes=False, use_tc_tiling_on_sc=False),
)(_sc_body)


def _finalize_body(sq_ref, w_ref, l1_ref, out_ref):
    dist = jnp.sqrt(sq_ref[...] + 1e-12)
    s2 = jnp.sum(dist * w_ref[...])
    out_ref[0] = l1_ref[0, 0] * (1.0 / (N * D)) + s2 * (1.0 / E)


def _finalize(sq, w, l1):
    return pl.pallas_call(
        _finalize_body,
        out_shape=jax.ShapeDtypeStruct((1,), jnp.float32),
        in_specs=[
            pl.BlockSpec(memory_space=pltpu.VMEM),
            pl.BlockSpec(memory_space=pltpu.VMEM),
            pl.BlockSpec(memory_space=pltpu.SMEM),
        ],
        out_specs=pl.BlockSpec(memory_space=pltpu.SMEM),
    )(sq.reshape(NCHUNK, CHUNK), w.reshape(NCHUNK, CHUNK), l1)


def kernel(x, edge_index, edge_weight, W_e, b_e, W_d, b_d):
    ybf, l1 = _encode(x, W_e, b_e, W_d, b_d)
    y_packed = lax.bitcast_convert_type(ybf.reshape(N, HW, 2), jnp.int32)
    row = edge_index[0]
    col = edge_index[1]
    sq = _sc_edge_sq(y_packed, row, col)
    out = _finalize(sq, edge_weight, l1)
    return out[0]
